# Initial kernel scaffold; baseline (speedup 1.0000x reference)
import jax
import jax.numpy as jnp
from jax.experimental import pallas as pl


def kernel(query, memory_stack, soh_constraint, soh_values, k):
    raise NotImplementedError("write your pallas kernel here")


# speedup vs baseline: 3.7110x; 3.7110x over previous
import functools

import jax
import jax.numpy as jnp
from jax import lax
from jax.experimental import pallas as pl
from jax.experimental.pallas import tpu as pltpu
from jax.experimental.pallas import tpu_sc as plsc

_SOH_TOL = 0.05
_NEG_INF = -1e30
_INIT = -1.0e38
_KILL = -3.0e38
_K = 16
_TM = 2048


def _counts_body(c_ref, sv_ref, cnt_ref):
    m = pl.program_id(0)

    @pl.when(m == 0)
    def _init():
        cnt_ref[...] = jnp.zeros_like(cnt_ref)

    sv = sv_ref[0, 0, :]
    valid = jnp.abs(c_ref[...] - sv[None, :]) <= _SOH_TOL
    cnt_ref[...] = cnt_ref[...] + jnp.sum(
        valid.astype(jnp.float32), axis=1, keepdims=True)


def _topk_body(c_ref, tol_ref, q_ref, mem_ref, sv_ref, idx_out_ref,
               cand_ref, rv_ref, ri_ref):
    m = pl.program_id(0)
    n_m = pl.num_programs(0)
    bsz = q_ref.shape[0]
    tile = mem_ref.shape[0]

    @pl.when(m == 0)
    def _init():
        rv_ref[...] = jnp.full((bsz, _K), _INIT, jnp.float32)
        ri_ref[...] = jnp.zeros((bsz, _K), jnp.int32)

    sim = lax.dot_general(q_ref[...], mem_ref[...], (((1,), (1,)), ((), ())),
                          preferred_element_type=jnp.float32)
    sv = sv_ref[0, 0, :]
    valid = jnp.abs(c_ref[...] - sv[None, :]) <= tol_ref[...]
    cand0 = jnp.where(valid, sim, _NEG_INF)
    cand_ref[...] = cand0

    cols = lax.broadcasted_iota(jnp.int32, (bsz, tile), 1)
    i16 = lax.broadcasted_iota(jnp.int32, (bsz, _K), 1)
    mx0 = jnp.max(cand0, axis=1, keepdims=True)
    first0 = jnp.min(jnp.where(cand0 == mx0, cols, tile),
                     axis=1, keepdims=True)
    base = m * tile

    def cond(carry):
        rv, _, mx, _ = carry
        return jnp.any(mx > rv[:, _K - 1:_K])

    def body(carry):
        rv, ri, mx, first = carry
        pos = jnp.sum((rv >= mx).astype(jnp.int32), axis=1, keepdims=True)
        sh_v = jnp.concatenate([rv[:, :1], rv[:, :_K - 1]], axis=1)
        sh_i = jnp.concatenate([ri[:, :1], ri[:, :_K - 1]], axis=1)
        nidx = base + first
        rv = jnp.where(i16 < pos, rv, jnp.where(i16 == pos, mx, sh_v))
        ri = jnp.where(i16 < pos, ri, jnp.where(i16 == pos, nidx, sh_i))
        cand = cand_ref[...]
        cand = jnp.where(cols == first, _KILL, cand)
        cand_ref[...] = cand
        mx2 = jnp.max(cand, axis=1, keepdims=True)
        first2 = jnp.min(jnp.where(cand == mx2, cols, tile),
                         axis=1, keepdims=True)
        return rv, ri, mx2, first2

    rv, ri, _, _ = lax.while_loop(
        cond, body, (rv_ref[...], ri_ref[...], mx0, first0))
    rv_ref[...] = rv
    ri_ref[...] = ri

    @pl.when(m == n_m - 1)
    def _fin():
        idx_out_ref[...] = ri


def _gather_rows(table, idx):
    bsz, kk = idx.shape
    _, dim = table.shape
    rows = (bsz * kk) // 128
    rpw = rows // 32
    idx2 = idx.reshape(rows, 128)
    mesh = plsc.VectorSubcoreMesh(core_axis_name="c", subcore_axis_name="s")

    @functools.partial(
        pl.kernel,
        out_type=jax.ShapeDtypeStruct((rows, 128, dim), jnp.float32),
        mesh=mesh,
        scratch_types=[
            pltpu.VMEM((rpw, 128), jnp.int32),
            pltpu.VMEM((rpw, 128, dim), jnp.float32),
            pltpu.SemaphoreType.DMA,
        ],
    )
    def gk(table_hbm, idx_hbm, out_hbm, idx_v, rows_v, sem):
        wid = lax.axis_index("s") * 2 + lax.axis_index("c")
        base = wid * rpw
        pltpu.sync_copy(idx_hbm.at[pl.ds(base, rpw)], idx_v)
        copies = [
            pltpu.async_copy(table_hbm.at[idx_v.at[i]], rows_v.at[i], sem)
            for i in range(rpw)
        ]
        for cp in copies:
            cp.wait()
        pltpu.sync_copy(rows_v, out_hbm.at[pl.ds(base, rpw)])

    return gk(table, idx2).reshape(bsz, kk, dim)


def kernel(query, memory_stack, soh_constraint, soh_values, k):
    bsz, dim = query.shape
    m_rows = memory_stack.shape[0]
    n_m = -(-m_rows // _TM)
    m_pad = n_m * _TM
    memp = jnp.pad(memory_stack, ((0, m_pad - m_rows), (0, 0)))
    svp = jnp.pad(soh_values, (0, m_pad - m_rows),
                  constant_values=2e9).reshape(n_m, 1, _TM)
    c2 = soh_constraint.reshape(bsz, 1)

    counts = pl.pallas_call(
        _counts_body,
        grid=(n_m,),
        in_specs=[pl.BlockSpec((bsz, 1), lambda m: (0, 0)),
                  pl.BlockSpec((1, 1, _TM), lambda m: (m, 0, 0))],
        out_specs=pl.BlockSpec((bsz, 1), lambda m: (0, 0)),
        out_shape=jax.ShapeDtypeStruct((bsz, 1), jnp.float32),
        compiler_params=pltpu.CompilerParams(
            dimension_semantics=("arbitrary",)),
    )(c2, svp)
    tol = jnp.where(counts < k, jnp.float32(_SOH_TOL * 2.0),
                    jnp.float32(_SOH_TOL))

    topk_idx = pl.pallas_call(
        _topk_body,
        grid=(n_m,),
        in_specs=[pl.BlockSpec((bsz, 1), lambda m: (0, 0)),
                  pl.BlockSpec((bsz, 1), lambda m: (0, 0)),
                  pl.BlockSpec((bsz, dim), lambda m: (0, 0)),
                  pl.BlockSpec((_TM, dim), lambda m: (m, 0)),
                  pl.BlockSpec((1, 1, _TM), lambda m: (m, 0, 0))],
        out_specs=pl.BlockSpec((bsz, _K), lambda m: (0, 0)),
        out_shape=jax.ShapeDtypeStruct((bsz, _K), jnp.int32),
        scratch_shapes=[pltpu.VMEM((bsz, _TM), jnp.float32),
                        pltpu.VMEM((bsz, _K), jnp.float32),
                        pltpu.VMEM((bsz, _K), jnp.int32)],
        compiler_params=pltpu.CompilerParams(
            dimension_semantics=("arbitrary",)),
    )(c2, tol, query, memp, svp)

    latents = _gather_rows(memory_stack, topk_idx)
    return latents, topk_idx
